# baseline (speedup 1.0000x reference)
import functools

import jax
import jax.numpy as jnp
from jax.experimental import pallas as pl
from jax.experimental.pallas import tpu as pltpu

S = 2048
D_MODEL = 1024
D_FF = 4096
E = 16
BLK = D_FF // E
S_TILE = 1024
N_S = S // S_TILE
FF_CHUNK = 512
N_F = D_FF // FF_CHUNK


def _ffn_kernel(h_ref, wr_ref, wg_ref, wu_ref, wd_ref, out_ref,
                wdb_ref, inter_ref, r_ref, g_ref, u_ref):
    f = pl.program_id(1)

    @pl.when(f < N_F)
    def _cast_wd():
        off = pl.multiple_of(f * FF_CHUNK, FF_CHUNK)
        wdb_ref[:, pl.ds(off, FF_CHUNK)] = wd_ref[...].astype(jnp.bfloat16)

    @pl.when(f == 0)
    def _router():
        hb = h_ref[...]
        logits = jax.lax.dot_general(
            hb, wr_ref[...].astype(jnp.bfloat16),
            dimension_numbers=(((1,), (1,)), ((), ())),
            preferred_element_type=jnp.float32,
        )
        r = jnp.maximum(logits, 0.0)
        r = r / (jnp.sum(r, axis=1, keepdims=True) + 1e-6)
        r_ref[...] = r.astype(jnp.bfloat16)

    @pl.when(f < N_F)
    def _gate_up():
        hb = h_ref[...]
        g_ref[f % 2] = jax.lax.dot_general(
            hb, wg_ref[...].astype(jnp.bfloat16),
            dimension_numbers=(((1,), (1,)), ((), ())),
            preferred_element_type=jnp.float32,
        )
        u_ref[f % 2] = jax.lax.dot_general(
            hb, wu_ref[...].astype(jnp.bfloat16),
            dimension_numbers=(((1,), (1,)), ((), ())),
            preferred_element_type=jnp.float32,
        )

    @pl.when(jnp.logical_and(f >= 1, f <= N_F))
    def _activate():
        p = f - 1
        col_expert = (
            jax.lax.broadcasted_iota(jnp.int32, (E, FF_CHUNK), 1) + p * FF_CHUNK
        ) // BLK
        row_expert = jax.lax.broadcasted_iota(jnp.int32, (E, FF_CHUNK), 0)
        onehot = (row_expert == col_expert).astype(jnp.bfloat16)
        scale = jax.lax.dot_general(
            r_ref[...], onehot, dimension_numbers=(((1,), (0,)), ((), ())),
            preferred_element_type=jnp.float32,
        )
        gate = g_ref[p % 2]
        up = u_ref[p % 2]
        inter = gate * jax.nn.sigmoid(gate) * up * scale
        offp = pl.multiple_of(p * FF_CHUNK, FF_CHUNK)
        inter_ref[:, pl.ds(offp, FF_CHUNK)] = inter.astype(jnp.bfloat16)

    @pl.when(f == N_F + 1)
    def _down():
        out_ref[...] = jax.lax.dot_general(
            inter_ref[...], wdb_ref[...],
            dimension_numbers=(((1,), (1,)), ((), ())),
            preferred_element_type=jnp.float32,
        )


@functools.partial(jax.jit, static_argnames=("interpret",))
def _run(h2d, wr, wg, wu, wd, interpret=False):
    last = N_F - 1

    def wgu_map(s, f):
        return (jnp.minimum(f, last), 0)

    def wd_map(s, f):
        return (0, jnp.minimum(f, last))

    out = pl.pallas_call(
        _ffn_kernel,
        grid=(N_S, N_F + 2),
        in_specs=[
            pl.BlockSpec((S_TILE, D_MODEL), lambda s, f: (s, 0)),
            pl.BlockSpec((E, D_MODEL), lambda s, f: (0, 0)),
            pl.BlockSpec((FF_CHUNK, D_MODEL), wgu_map),
            pl.BlockSpec((FF_CHUNK, D_MODEL), wgu_map),
            pl.BlockSpec((D_MODEL, FF_CHUNK), wd_map),
        ],
        out_specs=pl.BlockSpec((S_TILE, D_MODEL), lambda s, f: (s, 0)),
        out_shape=jax.ShapeDtypeStruct((S, D_MODEL), jnp.float32),
        scratch_shapes=[
            pltpu.VMEM((D_MODEL, D_FF), jnp.bfloat16),
            pltpu.VMEM((S_TILE, D_FF), jnp.bfloat16),
            pltpu.VMEM((S_TILE, E), jnp.bfloat16),
            pltpu.VMEM((2, S_TILE, FF_CHUNK), jnp.float32),
            pltpu.VMEM((2, S_TILE, FF_CHUNK), jnp.float32),
        ],
        compiler_params=pltpu.CompilerParams(
            dimension_semantics=("parallel", "arbitrary"),
            vmem_limit_bytes=66060288,
        ),
        interpret=interpret,
    )(h2d.astype(jnp.bfloat16), wr, wg, wu, wd)
    return out


def kernel(hidden_states, Wr, Wg, Wu, Wd):
    b, s, d = hidden_states.shape
    out = _run(hidden_states.reshape(s, d), Wr, Wg, Wu, Wd)
    return out.reshape(b, s, d)


# speedup vs baseline: 1.1168x; 1.1168x over previous
import functools

import jax
import jax.numpy as jnp
from jax.experimental import pallas as pl
from jax.experimental.pallas import tpu as pltpu

S = 2048
D_MODEL = 1024
D_FF = 4096
E = 16
BLK = D_FF // E
S_TILE = 1024
N_S = S // S_TILE
FF_CHUNK = 512
N_F = D_FF // FF_CHUNK


def _ffn_kernel(h_ref, wr_ref, wg_ref, wu_ref, wd_ref, out_ref,
                wdb_ref, inter_ref, rexp_ref, hb_ref):
    f = pl.program_id(1)
    off = pl.multiple_of(f * FF_CHUNK, FF_CHUNK)

    @pl.when(f < N_F)
    def _cast_wd():
        wdb_ref[:, pl.ds(off, FF_CHUNK)] = wd_ref[...].astype(jnp.bfloat16)

    @pl.when(f == 0)
    def _router():
        hb = h_ref[...].astype(jnp.bfloat16)
        hb_ref[...] = hb
        logits = jax.lax.dot_general(
            hb, wr_ref[...].astype(jnp.bfloat16),
            dimension_numbers=(((1,), (1,)), ((), ())),
            preferred_element_type=jnp.float32,
        )
        r = jnp.maximum(logits, 0.0)
        r = r / (jnp.sum(r, axis=1, keepdims=True) + 1e-6)
        rb = r.astype(jnp.bfloat16)
        rexp_ref[...] = jnp.concatenate(
            [jnp.broadcast_to(rb[:, e:e + 1], (S_TILE, BLK)) for e in range(E)],
            axis=1,
        )

    @pl.when(f < N_F)
    def _gate_up():
        hb = hb_ref[...]
        gate = jax.lax.dot_general(
            hb, wg_ref[...].astype(jnp.bfloat16),
            dimension_numbers=(((1,), (1,)), ((), ())),
            preferred_element_type=jnp.float32,
        )
        up = jax.lax.dot_general(
            hb, wu_ref[...].astype(jnp.bfloat16),
            dimension_numbers=(((1,), (1,)), ((), ())),
            preferred_element_type=jnp.float32,
        )
        scale = rexp_ref[:, pl.ds(off, FF_CHUNK)].astype(jnp.float32)
        inter = gate * jax.nn.sigmoid(gate) * up * scale
        inter_ref[:, pl.ds(off, FF_CHUNK)] = inter.astype(jnp.bfloat16)

    @pl.when(f == N_F)
    def _down():
        out_ref[...] = jax.lax.dot_general(
            inter_ref[...], wdb_ref[...],
            dimension_numbers=(((1,), (1,)), ((), ())),
            preferred_element_type=jnp.float32,
        )


@functools.partial(jax.jit, static_argnames=("interpret",))
def _run(h2d, wr, wg, wu, wd, interpret=False):
    last = N_F - 1

    def wgu_map(s, f):
        return (jnp.minimum(f, last), 0)

    def wd_map(s, f):
        return (0, jnp.minimum(f, last))

    out = pl.pallas_call(
        _ffn_kernel,
        grid=(N_S, N_F + 1),
        in_specs=[
            pl.BlockSpec((S_TILE, D_MODEL), lambda s, f: (s, 0)),
            pl.BlockSpec((E, D_MODEL), lambda s, f: (0, 0)),
            pl.BlockSpec((FF_CHUNK, D_MODEL), wgu_map),
            pl.BlockSpec((FF_CHUNK, D_MODEL), wgu_map),
            pl.BlockSpec((D_MODEL, FF_CHUNK), wd_map),
        ],
        out_specs=pl.BlockSpec((S_TILE, D_MODEL), lambda s, f: (s, 0)),
        out_shape=jax.ShapeDtypeStruct((S, D_MODEL), jnp.float32),
        scratch_shapes=[
            pltpu.VMEM((D_MODEL, D_FF), jnp.bfloat16),
            pltpu.VMEM((S_TILE, D_FF), jnp.bfloat16),
            pltpu.VMEM((S_TILE, D_FF), jnp.bfloat16),
            pltpu.VMEM((S_TILE, D_MODEL), jnp.bfloat16),
        ],
        compiler_params=pltpu.CompilerParams(
            dimension_semantics=("parallel", "arbitrary"),
            vmem_limit_bytes=66060288,
        ),
        interpret=interpret,
    )(h2d, wr, wg, wu, wd)
    return out


def kernel(hidden_states, Wr, Wg, Wu, Wd):
    b, s, d = hidden_states.shape
    out = _run(hidden_states.reshape(s, d), Wr, Wg, Wu, Wd)
    return out.reshape(b, s, d)
